# baseline (speedup 1.0000x reference)
import jax
import jax.numpy as jnp
from jax.experimental import pallas as pl

_B = 16384
_Q = 512
_E = 64
_BLK = 4096


def _fused_kernel(x_ref, kdw_ref, cbw_ref, out_ref):
    xt = x_ref[...].T

    kdt = jax.lax.dot_general(
        kdw_ref[...], xt, (((1,), (0,)), ((), ())),
        preferred_element_type=jnp.float32,
    )
    kd_max = jnp.max(kdt, axis=0, keepdims=True)
    col = jax.lax.broadcasted_iota(jnp.int32, kdt.shape, 0)
    k_idx = jnp.min(jnp.where(kdt == kd_max, col, _E), axis=0, keepdims=True)
    k = k_idx + 1
    kf = k.astype(jnp.float32)

    ub = jax.lax.bitcast_convert_type(xt, jnp.uint32)
    topbit = jnp.uint32(0x80000000)
    ub = jnp.where(ub == topbit, jnp.uint32(0), ub)
    u = jnp.where(ub >= topbit, ~ub, ub | topbit)

    zero_row = jnp.zeros_like(u[:1, :])
    ones_q = jnp.ones((1, _Q), jnp.float32)

    def cond_fn(state):
        bit, _, _, ndone = state
        return (bit >= 0) & (ndone > 0)

    def body_fn(state):
        bit, p, hitcand, _ = state
        cand = p | (jnp.uint32(1) << jnp.uint32(bit))
        mf = jnp.where(u >= cand, 1.0, 0.0)
        c = jnp.sum(mf, axis=0, keepdims=True)
        p = jnp.where(c >= kf, cand, p)
        hit = (c == kf) & (hitcand == 0)
        hitcand = jnp.where(hit, cand, hitcand)
        ndone = jnp.sum(jnp.where(hitcand == 0, 1.0, 0.0))
        return bit - 1, p, hitcand, ndone

    _, p_final, hitcand, ndone = jax.lax.while_loop(
        cond_fn, body_fn, (31, zero_row, zero_row, jnp.float32(1.0)))

    def no_ties(_):
        return jnp.where(u >= hitcand, 1.0, 0.0)

    def with_ties(_):
        thr = jnp.where(hitcand == 0, p_final, hitcand)
        gt = jnp.where(u > thr, 1.0, 0.0)
        need = k - jnp.sum(gt, axis=0, keepdims=True).astype(jnp.int32)
        idx = jax.lax.broadcasted_iota(jnp.int32, u.shape, 0)
        eq = (u == thr)

        def idx_step(i, p):
            cand = p + (1 << (9 - i))
            sel = jnp.where(eq & (idx < cand), 1.0, 0.0)
            c = jnp.sum(sel, axis=0, keepdims=True).astype(jnp.int32)
            return jnp.where(c <= need, cand, p)

        pidx = jax.lax.fori_loop(0, 10, idx_step, jnp.zeros_like(k))
        tie_mask = gt + jnp.where(eq & (idx < pidx), 1.0, 0.0)
        exact = jnp.where(u >= hitcand, 1.0, 0.0)
        return jnp.where(hitcand == 0, tie_mask, exact)

    k_hot = jax.lax.cond(ndone == 0, no_ties, with_ties, operand=None)

    out_t = jax.lax.dot_general(
        cbw_ref[...], k_hot, (((1,), (0,)), ((), ())),
        preferred_element_type=jnp.float32,
    )
    out_ref[...] = out_t.T


@jax.jit
def kernel(x, codebook_weight, k_decider_weight):
    grid = (_B // _BLK,)
    return pl.pallas_call(
        _fused_kernel,
        grid=grid,
        in_specs=[
            pl.BlockSpec((_BLK, _Q), lambda i: (i, 0)),
            pl.BlockSpec((_E, _Q), lambda i: (0, 0)),
            pl.BlockSpec((_E, _Q), lambda i: (0, 0)),
        ],
        out_specs=pl.BlockSpec((_BLK, _E), lambda i: (i, 0)),
        out_shape=jax.ShapeDtypeStruct((_B, _E), jnp.float32),
    )(x, k_decider_weight, codebook_weight)


# speedup vs baseline: 1.2956x; 1.2956x over previous
import jax
import jax.numpy as jnp
from jax.experimental import pallas as pl

_B = 16384
_Q = 512
_E = 64
_BLK = 4096


def _fused_kernel(x_ref, kdw_ref, cbw_ref, out_ref):
    xt = x_ref[...].T

    kdt = jax.lax.dot_general(
        kdw_ref[...], xt, (((1,), (0,)), ((), ())),
        preferred_element_type=jnp.float32,
    )
    kd_max = jnp.max(kdt, axis=0, keepdims=True)
    col = jax.lax.broadcasted_iota(jnp.int32, kdt.shape, 0)
    k_idx = jnp.min(jnp.where(kdt == kd_max, col, _E), axis=0, keepdims=True)
    k = k_idx + 1
    kf = k.astype(jnp.float32)

    ub = jax.lax.bitcast_convert_type(xt, jnp.uint32)
    topbit = jnp.uint32(0x80000000)
    ub = jnp.where(ub == topbit, jnp.uint32(0), ub)
    u = jnp.where(ub >= topbit, ~ub, ub | topbit)

    zero_row = jnp.zeros_like(u[:1, :])
    ones_bf = jnp.ones((1, _Q), jnp.bfloat16)
    one_bf = jnp.bfloat16(1.0)
    zero_bf = jnp.bfloat16(0.0)
    vhi = ((u >> jnp.uint32(16)).astype(jnp.int32) - 32768).astype(jnp.int16)

    def cond_fn(state):
        bit, _, _, ndone = state
        return (bit >= 0) & (ndone > 0)

    def hi_body(state):
        bit, p, hitcand, _ = state
        cand = p | (1 << bit)
        cand16 = (cand - 32768).astype(jnp.int16)
        mf = jnp.where(vhi >= cand16, one_bf, zero_bf)
        c = jax.lax.dot_general(
            ones_bf, mf, (((1,), (0,)), ((), ())),
            preferred_element_type=jnp.float32)
        p = jnp.where(c >= kf, cand, p)
        hit = (c == kf) & (hitcand == 0)
        hitcand = jnp.where(hit, cand.astype(jnp.uint32) << jnp.uint32(16),
                            hitcand)
        ndone = jnp.sum(jnp.where(hitcand == 0, 1.0, 0.0))
        return bit - 1, p, hitcand, ndone

    _, p_hi, hitcand, ndone = jax.lax.while_loop(
        cond_fn, hi_body, (15, jnp.zeros_like(u[:1, :], jnp.int32),
                           zero_row, jnp.float32(1.0)))

    def lo_body(state):
        bit, p, hitcand, _ = state
        cand = p | (jnp.uint32(1) << jnp.uint32(bit))
        mf = jnp.where(u >= cand, 1.0, 0.0)
        c = jax.lax.dot_general(
            jnp.ones((1, _Q), jnp.float32), mf, (((1,), (0,)), ((), ())),
            preferred_element_type=jnp.float32)
        p = jnp.where(c >= kf, cand, p)
        hit = (c == kf) & (hitcand == 0)
        hitcand = jnp.where(hit, cand, hitcand)
        ndone = jnp.sum(jnp.where(hitcand == 0, 1.0, 0.0))
        return bit - 1, p, hitcand, ndone

    _, p_final, hitcand, ndone = jax.lax.while_loop(
        cond_fn, lo_body,
        (15, p_hi.astype(jnp.uint32) << jnp.uint32(16), hitcand, ndone))

    def no_ties(_):
        return jnp.where(u >= hitcand, 1.0, 0.0)

    def with_ties(_):
        thr = jnp.where(hitcand == 0, p_final, hitcand)
        gt = jnp.where(u > thr, 1.0, 0.0)
        need = k - jnp.sum(gt, axis=0, keepdims=True).astype(jnp.int32)
        idx = jax.lax.broadcasted_iota(jnp.int32, u.shape, 0)
        eq = (u == thr)

        def idx_step(i, p):
            cand = p + (1 << (9 - i))
            sel = jnp.where(eq & (idx < cand), 1.0, 0.0)
            c = jnp.sum(sel, axis=0, keepdims=True).astype(jnp.int32)
            return jnp.where(c <= need, cand, p)

        pidx = jax.lax.fori_loop(0, 10, idx_step, jnp.zeros_like(k))
        tie_mask = gt + jnp.where(eq & (idx < pidx), 1.0, 0.0)
        exact = jnp.where(u >= hitcand, 1.0, 0.0)
        return jnp.where(hitcand == 0, tie_mask, exact)

    k_hot = jax.lax.cond(ndone == 0, no_ties, with_ties, operand=None)

    out_t = jax.lax.dot_general(
        cbw_ref[...], k_hot, (((1,), (0,)), ((), ())),
        preferred_element_type=jnp.float32,
    )
    out_ref[...] = out_t.T


@jax.jit
def kernel(x, codebook_weight, k_decider_weight):
    grid = (_B // _BLK,)
    return pl.pallas_call(
        _fused_kernel,
        grid=grid,
        in_specs=[
            pl.BlockSpec((_BLK, _Q), lambda i: (i, 0)),
            pl.BlockSpec((_E, _Q), lambda i: (0, 0)),
            pl.BlockSpec((_E, _Q), lambda i: (0, 0)),
        ],
        out_specs=pl.BlockSpec((_BLK, _E), lambda i: (i, 0)),
        out_shape=jax.ShapeDtypeStruct((_B, _E), jnp.float32),
    )(x, k_decider_weight, codebook_weight)
